# baseline (speedup 1.0000x reference)
import functools

import jax
import jax.numpy as jnp
from jax import lax
from jax.experimental import pallas as pl
from jax.experimental.pallas import tpu as pltpu
from jax.experimental.pallas import tpu_sc as plsc

N = 10000
E = 320000
D = 128
H = 64
C = 40
K = 16
ALPHA = 0.05

NS = 16
FA = 24
FB = 16
F = FA + FB
R = 640
NP = NS * R
CH = 128
NCHUNK = -(-E // (NS * CH))
EPT = NCHUNK * CH
EPAD = EPT * NS
NSUB = R // CH

_mesh2 = plsc.VectorSubcoreMesh(
    core_axis_name="c", subcore_axis_name="s", num_cores=2
)
_sc_params = pltpu.CompilerParams(use_tc_tiling_on_sc=False)


@functools.partial(
    pl.kernel,
    out_type=(
        jax.ShapeDtypeStruct((NP, FA), jnp.float32),
        jax.ShapeDtypeStruct((NP, FB), jnp.float32),
        jax.ShapeDtypeStruct((NP,), jnp.float32),
    ),
    mesh=_mesh2,
    scratch_types=[
        pltpu.VMEM_SHARED((NP, FA), jnp.float32),
        pltpu.VMEM_SHARED((NP, FA), jnp.float32),
        pltpu.VMEM_SHARED((NP, FB), jnp.float32),
        pltpu.VMEM_SHARED((NP, FB), jnp.float32),
        pltpu.VMEM_SHARED((NP,), jnp.float32),
        pltpu.VMEM((NCHUNK, CH), jnp.int32),
        pltpu.VMEM((NCHUNK, CH), jnp.int32),
        pltpu.VMEM((CH, FA), jnp.float32),
        pltpu.VMEM((CH, FA), jnp.float32),
        pltpu.VMEM((CH, FA), jnp.float32),
        pltpu.VMEM((CH, FA), jnp.float32),
        pltpu.VMEM((CH, FB), jnp.float32),
        pltpu.VMEM((CH, FB), jnp.float32),
        pltpu.VMEM((CH, FB), jnp.float32),
        pltpu.VMEM((CH, FB), jnp.float32),
        pltpu.VMEM((R,), jnp.float32),
        pltpu.VMEM((R,), jnp.float32),
        pltpu.VMEM((CH,), jnp.float32),
        pltpu.SemaphoreType.DMA,
        pltpu.SemaphoreType.DMA,
    ],
    compiler_params=_sc_params,
)
def _prop_kernel(
    ha_hbm, hb_hbm, rows_hbm, cols_hbm, va_hbm, vb_hbm, dinv_hbm,
    va_sp, wa_sp, vb_sp, wb_sp, deg_sp, row_t, col_t,
    ga, ua, ga2, ua2, gb, ub, gb2, ub2, dbuf, disb, ones_v, gsem, ssem,
):
    ci = lax.axis_index("c")
    t = lax.axis_index("s")
    sl = pl.ds(t * R, R)
    z16 = jnp.zeros((16,), jnp.float32)

    pltpu.sync_copy(rows_hbm.at[t], row_t)
    pltpu.sync_copy(cols_hbm.at[t], col_t)

    def _vslices(fw):
        out = [(j * 16, 16) for j in range(fw // 16)]
        if fw % 16:
            out.append((fw - fw % 16, fw % 16))
        return out

    def fill0(i, _):
        dbuf[pl.ds(i * 16, 16)] = z16
        return 0

    lax.fori_loop(0, R // 16, fill0, 0)

    def fill1(i, _):
        ones_v[pl.ds(i * 16, 16)] = jnp.ones((16,), jnp.float32)
        return 0

    lax.fori_loop(0, CH // 16, fill1, 0)
    pltpu.sync_copy(dbuf, deg_sp.at[sl])
    plsc.subcore_barrier()

    def dchunk(c, _):
        pltpu.sync_copy(ones_v, deg_sp.at[row_t.at[c]], add=True)
        return 0

    lax.fori_loop(0, NCHUNK, dchunk, 0)
    plsc.subcore_barrier()

    pltpu.sync_copy(deg_sp.at[sl], dbuf)

    def dcalc(i, _):
        x = dbuf[pl.ds(i * 16, 16)] + 1.0
        xi = lax.bitcast_convert_type(x, jnp.int32)
        yi = jnp.int32(0x5F3759DF) - lax.shift_right_arithmetic(xi, 1)
        y = lax.bitcast_convert_type(yi, jnp.float32)
        hx = 0.5 * x
        for _ in range(4):
            y = y * (1.5 - hx * y * y)
        disb[pl.ds(i * 16, 16)] = y
        dbuf[pl.ds(i * 16, 16)] = y * y
        return 0

    lax.fori_loop(0, R // 16, dcalc, 0)

    @pl.when(ci == 0)
    def _():
        pltpu.sync_copy(dbuf, dinv_hbm.at[sl])

    def _init(fw, h_hbm, v_sp, w_sp, gbuf, ubuf):
        def zrow(r, _):
            for (o, n) in _vslices(fw):
                ubuf[r, pl.ds(o, n)] = jnp.zeros((n,), jnp.float32)
            return 0

        lax.fori_loop(0, CH, zrow, 0)

        for s in range(NSUB):
            base = pl.ds(t * R + s * CH, CH)
            pltpu.sync_copy(h_hbm.at[base], gbuf)

            def scale(i, _):
                d16 = disb[pl.ds(s * CH + i * 16, 16)]
                for l in range(16):
                    r = i * 16 + l
                    d = d16[l]
                    for (o, n) in _vslices(fw):
                        gbuf[r, pl.ds(o, n)] = d * gbuf[r, pl.ds(o, n)]
                return 0

            lax.fori_loop(0, CH // 16, scale, 0)
            pltpu.sync_copy(gbuf, v_sp.at[base])
            pltpu.sync_copy(ubuf, w_sp.at[base])

    @pl.when(ci == 0)
    def _():
        _init(FA, ha_hbm, va_sp, wa_sp, ga, ua)

    @pl.when(ci == 1)
    def _():
        _init(FB, hb_hbm, vb_sp, wb_sp, gb, ub)

    plsc.subcore_barrier()

    def _run_chunks(v_sp, w_sp, g0, g1, dummy):
        pltpu.async_copy(v_sp.at[row_t.at[0]], g0, gsem)

        def pair(p, _):
            c0 = 2 * p
            pltpu.async_copy(v_sp.at[row_t.at[c0 + 1]], g1, gsem)
            pltpu.make_async_copy(dummy, g0, gsem).wait()
            pltpu.sync_copy(g0, w_sp.at[col_t.at[c0]], add=True)
            pltpu.async_copy(v_sp.at[row_t.at[c0 + 2]], g0, gsem)
            pltpu.make_async_copy(dummy, g1, gsem).wait()
            pltpu.sync_copy(g1, w_sp.at[col_t.at[c0 + 1]], add=True)
            return 0

        lax.fori_loop(0, NCHUNK // 2, pair, 0)
        pltpu.make_async_copy(dummy, g0, gsem).wait()
        pltpu.sync_copy(g0, w_sp.at[col_t.at[NCHUNK - 1]], add=True)

    def _upd_block(fw, s, gbuf, ubuf):
        def upd(i, _):
            d16 = dbuf[pl.ds(s * CH + i * 16, 16)]
            for l in range(16):
                r = i * 16 + l
                d = d16[l]
                sc = 1.0 + d
                for (o, n) in _vslices(fw):
                    ds_ = pl.ds(o, n)
                    gbuf[r, ds_] = sc * gbuf[r, ds_] + d * ubuf[r, ds_]
                    ubuf[r, ds_] = jnp.zeros((n,), jnp.float32)
            return 0

        lax.fori_loop(0, CH // 16, upd, 0)

    def _update_all(fw, v_sp, w_sp, bufs, dummy):
        def base(s):
            return pl.ds(t * R + s * CH, CH)

        def drain(sem, gbuf):
            pltpu.make_async_copy(dummy, gbuf, sem).wait()
            pltpu.make_async_copy(dummy, gbuf, sem).wait()

        pltpu.async_copy(v_sp.at[base(0)], bufs[0][0], gsem)
        pltpu.async_copy(w_sp.at[base(0)], bufs[0][1], gsem)
        for s in range(NSUB):
            g_, u_ = bufs[s % 2]
            if s >= 1:
                drain(ssem, g_)
            if s + 1 < NSUB:
                gn, un = bufs[(s + 1) % 2]
                pltpu.async_copy(v_sp.at[base(s + 1)], gn, gsem)
                pltpu.async_copy(w_sp.at[base(s + 1)], un, gsem)
            drain(gsem, g_)
            _upd_block(fw, s, g_, u_)
            pltpu.async_copy(g_, v_sp.at[base(s)], ssem)
            pltpu.async_copy(u_, w_sp.at[base(s)], ssem)
        drain(ssem, bufs[(NSUB - 1) % 2][0])

    def hop(k, _):
        @pl.when(ci == 0)
        def _():
            _run_chunks(va_sp, wa_sp, ga, ua, ha_hbm.at[pl.ds(0, CH)])

        @pl.when(ci == 1)
        def _():
            _run_chunks(vb_sp, wb_sp, gb, ub, hb_hbm.at[pl.ds(0, CH)])

        plsc.subcore_barrier()

        @pl.when(ci == 0)
        def _():
            _update_all(FA, va_sp, wa_sp, ((ga, ua), (ga2, ua2)),
                        ha_hbm.at[pl.ds(0, CH)])

        @pl.when(ci == 1)
        def _():
            _update_all(FB, vb_sp, wb_sp, ((gb, ub), (gb2, ub2)),
                        hb_hbm.at[pl.ds(0, CH)])

        plsc.subcore_barrier()
        return 0

    lax.fori_loop(0, K, hop, 0)

    @pl.when(ci == 0)
    def _():
        pltpu.sync_copy(va_sp.at[sl], va_hbm.at[sl])

    @pl.when(ci == 1)
    def _():
        pltpu.sync_copy(vb_sp.at[sl], vb_hbm.at[sl])


def _head_body(x_ref, w1_ref, b1_ref, w2_ref, b2_ref,
               h_ref, ha_ref, hb_ref):
    hmid = jnp.maximum(
        jnp.dot(x_ref[...], w1_ref[...], preferred_element_type=jnp.float32)
        + b1_ref[...],
        0.0,
    )
    h = jnp.dot(hmid, w2_ref[...], preferred_element_type=jnp.float32) + b2_ref[...]
    if F > C:
        hp = jnp.concatenate([h, jnp.zeros((NP, F - C), jnp.float32)], axis=1)
    else:
        hp = h
    h_ref[...] = hp
    ha_ref[...] = hp[:, :FA]
    hb_ref[...] = hp[:, FA:]


def _head(xp, W1, b1, W2, b2):
    return pl.pallas_call(
        _head_body,
        out_shape=(
            jax.ShapeDtypeStruct((NP, F), jnp.float32),
            jax.ShapeDtypeStruct((NP, FA), jnp.float32),
            jax.ShapeDtypeStruct((NP, FB), jnp.float32),
        ),
    )(xp, W1, b1.reshape(1, H), W2, b2.reshape(1, C))


def _tail_body(h_ref, va_ref, vb_ref, di_ref, o_ref):
    sq = lax.rsqrt(di_ref[pl.ds(0, N), :])
    h = h_ref[pl.ds(0, N), pl.ds(0, C)]
    v = jnp.concatenate(
        [va_ref[pl.ds(0, N), :], vb_ref[pl.ds(0, N), pl.ds(0, C - FA)]], axis=1
    )
    res = ALPHA * h + ((1.0 - ALPHA) / K) * sq * v
    m = jnp.max(res, axis=1, keepdims=True)
    ex = jnp.exp(res - m)
    lse = jnp.log(jnp.sum(ex, axis=1, keepdims=True))
    o_ref[...] = res - m - lse


def _tail(h_pad, vKa, vKb, deg_inv):
    return pl.pallas_call(
        _tail_body,
        out_shape=jax.ShapeDtypeStruct((N, C), jnp.float32),
    )(h_pad, vKa, vKb, deg_inv)


def kernel(x, edge_index, W1, b1, W2, b2):
    pad_e = EPAD - E
    rows = jnp.concatenate(
        [edge_index[0], jnp.full((pad_e,), N, jnp.int32)]
    ).reshape(NS, NCHUNK, CH)
    cols = jnp.concatenate(
        [edge_index[1], jnp.full((pad_e,), N, jnp.int32)]
    ).reshape(NS, NCHUNK, CH)
    xp = jnp.pad(x, ((0, NP - N), (0, 0)))

    h_pad, ha, hb = _head(xp, W1, b1, W2, b2)
    vKa, vKb, deg_inv = _prop_kernel(ha, hb, rows, cols)
    return _tail(h_pad, vKa, vKb, deg_inv.reshape(NP, 1))
